# baseline (speedup 1.0000x reference)
import functools

import jax
import jax.numpy as jnp
from jax import lax
from jax.experimental import pallas as pl
from jax.experimental.pallas import tpu as pltpu, tpu_sc as plsc

N_NODES = 10000
N_EDGES = 160000
F_IN = 128
HID = 32
HEADS = 2
T = 8

N_PAD = 10240
E_REAL = N_EDGES + N_NODES
E_PAD = 172032
EC = E_PAD // 32
NJ = EC // 16
TN = T * N_NODES


_BLK_A = 3200


def _tcA_body(x_ref, wge_ref, wgo_ref, a8e_ref, a8o_ref, pk_ref, al_ref):
    xb = x_ref[...]
    hlo = lax.dot_general(wge_ref[...], xb, (((0,), (1,)), ((), ())),
                          preferred_element_type=jnp.float32)
    hhi = lax.dot_general(wgo_ref[...], xb, (((0,), (1,)), ((), ())),
                          preferred_element_type=jnp.float32)
    al = (jnp.dot(a8e_ref[...], hlo, preferred_element_type=jnp.float32)
          + jnp.dot(a8o_ref[...], hhi,
                    preferred_element_type=jnp.float32))
    alb = lax.bitcast_convert_type(al.astype(jnp.bfloat16),
                                   jnp.uint16).astype(jnp.uint32)
    pk_s = jnp.bitwise_or(alb[0:1], jnp.left_shift(alb[1:2], 16))
    pk_d = jnp.bitwise_or(alb[2:3], jnp.left_shift(alb[3:4], 16))
    al_ref[...] = lax.bitcast_convert_type(
        jnp.concatenate([pk_s, pk_d], axis=0), jnp.int32)
    blo = lax.bitcast_convert_type(hlo.astype(jnp.bfloat16),
                                   jnp.uint16).astype(jnp.uint32)
    bhi = lax.bitcast_convert_type(hhi.astype(jnp.bfloat16),
                                   jnp.uint16).astype(jnp.uint32)
    pk_ref[...] = lax.bitcast_convert_type(
        jnp.bitwise_or(blo, jnp.left_shift(bhi, 16)), jnp.int32)


def _tcA(x2d, Wg_even, Wg_odd, A8_even, A8_odd):
    return pl.pallas_call(
        _tcA_body,
        grid=(TN // _BLK_A,),
        in_specs=[
            pl.BlockSpec((_BLK_A, F_IN), lambda i: (i, 0)),
            pl.BlockSpec((F_IN, HID), lambda i: (0, 0)),
            pl.BlockSpec((F_IN, HID), lambda i: (0, 0)),
            pl.BlockSpec((8, HID), lambda i: (0, 0)),
            pl.BlockSpec((8, HID), lambda i: (0, 0)),
        ],
        out_specs=[
            pl.BlockSpec((HID, _BLK_A), lambda i: (0, i)),
            pl.BlockSpec((2, _BLK_A), lambda i: (0, i)),
        ],
        out_shape=[
            jax.ShapeDtypeStruct((HID, TN), jnp.int32),
            jax.ShapeDtypeStruct((2, TN), jnp.int32),
        ],
    )(x2d, Wg_even, Wg_odd, A8_even, A8_odd)


def _sc_b0_body(alp, ep_h, w_h, s01, d01, epv, w0, w1):
    wid = lax.axis_index("s") * 2 + lax.axis_index("c")
    base = wid * EC
    pltpu.sync_copy(ep_h.at[pl.ds(base, EC)], epv)

    izeros16 = jnp.zeros((16,), jnp.int32)

    def zero_tail(k, _):
        off = N_NODES + k * 16
        s01[pl.ds(off, 16)] = izeros16
        d01[pl.ds(off, 16)] = izeros16
        return 0

    lax.fori_loop(0, (N_PAD - N_NODES) // 16, zero_tail, 0)

    def t_loop(t, _):
        pltpu.sync_copy(alp.at[pl.ds(0 * TN + t * N_NODES, N_NODES)],
                        s01.at[pl.ds(0, N_NODES)])
        pltpu.sync_copy(alp.at[pl.ds(1 * TN + t * N_NODES, N_NODES)],
                        d01.at[pl.ds(0, N_NODES)])

        def j_body(j):
            ev = epv[pl.ds(j * 16, 16)]
            vs = jnp.bitwise_and(ev, 0x3FFF)
            vd = lax.shift_right_logical(ev, 14)
            gs = plsc.load_gather(s01, [vs])
            gd = plsc.load_gather(d01, [vd])
            e0 = (plsc.bitcast(jnp.left_shift(gs, 16), jnp.float32)
                  + plsc.bitcast(jnp.left_shift(gd, 16), jnp.float32))
            e1 = (plsc.bitcast(jnp.bitwise_and(gs, -65536), jnp.float32)
                  + plsc.bitcast(jnp.bitwise_and(gd, -65536), jnp.float32))
            e0 = jnp.where(e0 >= 0.0, e0, 0.2 * e0)
            e1 = jnp.where(e1 >= 0.0, e1, 0.2 * e1)
            w0[pl.ds(j * 16, 16)] = jnp.exp(e0)
            w1[pl.ds(j * 16, 16)] = jnp.exp(e1)

        plsc.parallel_loop(0, NJ, 1, unroll=8)(j_body)
        pltpu.sync_copy(w0, w_h.at[pl.ds((t * 2 + 0) * E_PAD + base, EC)])
        pltpu.sync_copy(w1, w_h.at[pl.ds((t * 2 + 1) * E_PAD + base, EC)])
        return 0

    lax.fori_loop(0, T, t_loop, 0)


@functools.lru_cache(maxsize=None)
def _build_sc_b0():
    mesh = plsc.VectorSubcoreMesh(core_axis_name="c", subcore_axis_name="s")
    return pl.kernel(
        _sc_b0_body,
        mesh=mesh,
        compiler_params=pltpu.CompilerParams(needs_layout_passes=False),
        out_type=jax.ShapeDtypeStruct((T * HEADS * E_PAD,), jnp.float32),
        scratch_types=[
            pltpu.VMEM((N_PAD,), jnp.int32),
            pltpu.VMEM((N_PAD,), jnp.int32),
            pltpu.VMEM((EC,), jnp.int32),
            pltpu.VMEM((EC,), jnp.float32),
            pltpu.VMEM((EC,), jnp.float32),
        ],
    )


_SEG = N_PAD // 16


def _sc_b1_body(hpk, w_h, ep_h, agg,
                tbl, acc0, acc1, accd, ep2, w2, shr, redbuf, dout,
                semA, semB):
    cid = lax.axis_index("c")
    sid = lax.axis_index("s")
    head = cid
    pk_row = cid * 16 + sid
    col0 = 2 * pk_row
    zeros16 = jnp.zeros((16,), jnp.float32)
    izeros16 = jnp.zeros((16,), jnp.int32)

    def t_loop(t, _):
        woff = (t * 2 + head) * E_PAD

        def issue(b, par, sem):
            pltpu.async_copy(ep_h.at[pl.ds(b * EC, EC)],
                             ep2.at[pl.ds(par * EC, EC)], sem)
            pltpu.async_copy(w_h.at[pl.ds(woff + b * EC, EC)],
                             w2.at[pl.ds(par * EC, EC)], sem)

        def drain(par, sem):
            pltpu.make_async_copy(ep_h.at[pl.ds(0, EC)],
                                  ep2.at[pl.ds(par * EC, EC)], sem).wait()
            pltpu.make_async_copy(w_h.at[pl.ds(0, EC)],
                                  w2.at[pl.ds(par * EC, EC)], sem).wait()

        issue(0, 0, semA)
        pltpu.sync_copy(hpk.at[pl.ds(pk_row * TN + t * N_NODES, N_NODES)],
                        tbl.at[pl.ds(0, N_NODES)])

        def zero_acc(k):
            acc0[pl.ds(k * 16, 16)] = zeros16
            acc1[pl.ds(k * 16, 16)] = zeros16
            accd[pl.ds(k * 16, 16)] = zeros16

        plsc.parallel_loop(0, N_PAD // 16, 1, unroll=4)(zero_acc)

        def zero_tail(k):
            tbl[pl.ds(N_NODES + k * 16, 16)] = izeros16

        plsc.parallel_loop(0, (N_PAD - N_NODES) // 16, 1,
                           unroll=2)(zero_tail)

        def make_j_body(par, with_den):
            def j_body(j):
                off = par * EC + j * 16
                ev = ep2[pl.ds(off, 16)]
                vs = jnp.bitwise_and(ev, 0x3FFF)
                vd = lax.shift_right_logical(ev, 14)
                wv = w2[pl.ds(off, 16)]
                g = plsc.load_gather(tbl, [vs])
                glo = plsc.bitcast(jnp.left_shift(g, 16), jnp.float32)
                ghi = plsc.bitcast(jnp.bitwise_and(g, -65536),
                                   jnp.float32)
                plsc.addupdate_scatter(acc0, [vd], wv * glo)
                plsc.addupdate_scatter(acc1, [vd], wv * ghi)
                if with_den:
                    plsc.addupdate_scatter(accd, [vd], wv)
            return j_body

        def process(par, b):
            do_den = lax.rem(b, 16) == sid

            @pl.when(do_den)
            def _():
                plsc.parallel_loop(0, NJ, 1, unroll=16)(make_j_body(par, True))

            @pl.when(jnp.logical_not(do_den))
            def _():
                plsc.parallel_loop(0, NJ, 1,
                                   unroll=16)(make_j_body(par, False))

        def bb_loop(bb, _):
            b0 = 2 * bb
            issue(b0 + 1, 1, semB)
            drain(0, semA)
            process(0, b0)

            @pl.when(bb < 15)
            def _():
                issue(b0 + 2, 0, semA)

            drain(1, semB)
            process(1, b0 + 1)
            return 0

        lax.fori_loop(0, 16, bb_loop, 0)

        pltpu.sync_copy(acc0, agg.at[pl.ds((col0 * T + t) * N_PAD, N_PAD)])
        pltpu.sync_copy(acc1, agg.at[pl.ds(((col0 + 1) * T + t) * N_PAD,
                                           N_PAD)])

        pltpu.sync_copy(accd, shr.at[pl.ds(sid * N_PAD, N_PAD)])
        plsc.subcore_barrier()
        for p in range(16):
            pltpu.sync_copy(shr.at[pl.ds(p * N_PAD + sid * _SEG, _SEG)],
                            redbuf.at[pl.ds(p * _SEG, _SEG)])

        def red_loop(k):
            s = redbuf[pl.ds(k * 16, 16)]
            for p in range(1, 16):
                s = s + redbuf[pl.ds(p * _SEG + k * 16, 16)]
            dout[pl.ds(k * 16, 16)] = s

        plsc.parallel_loop(0, _SEG // 16, 1, unroll=2)(red_loop)
        pltpu.sync_copy(
            dout,
            agg.at[pl.ds(((64 + head) * T + t) * N_PAD + sid * _SEG, _SEG)])
        plsc.subcore_barrier()
        return 0

    lax.fori_loop(0, T, t_loop, 0)


@functools.lru_cache(maxsize=None)
def _build_sc_b1():
    mesh = plsc.VectorSubcoreMesh(core_axis_name="c", subcore_axis_name="s")
    return pl.kernel(
        _sc_b1_body,
        mesh=mesh,
        compiler_params=pltpu.CompilerParams(needs_layout_passes=False),
        out_type=jax.ShapeDtypeStruct((66 * T * N_PAD,), jnp.float32),
        scratch_types=[
            pltpu.VMEM((N_PAD,), jnp.int32),
            pltpu.VMEM((N_PAD,), jnp.float32),
            pltpu.VMEM((N_PAD,), jnp.float32),
            pltpu.VMEM((N_PAD,), jnp.float32),
            pltpu.VMEM((2 * EC,), jnp.int32),
            pltpu.VMEM((2 * EC,), jnp.float32),
            pltpu.VMEM_SHARED((16 * N_PAD,), jnp.float32),
            pltpu.VMEM((16 * _SEG,), jnp.float32),
            pltpu.VMEM((_SEG,), jnp.float32),
            pltpu.SemaphoreType.DMA,
            pltpu.SemaphoreType.DMA,
        ],
    )


_BLK_B = 1280


def _tcB_body(agg_ref, wih_ref, whh_ref, bg_ref, wa1_ref, ba1_ref, wa2_ref,
              ba2_ref, wd1_ref, bd1_ref, wd2_ref, bd2_ref, bgat_ref, out_ref):
    wih = wih_ref[...]
    whh = whh_ref[...]
    bg = bg_ref[...]
    bgat = bgat_ref[...]

    h = jnp.zeros((HID, _BLK_B), jnp.float32)
    c = jnp.zeros((HID, _BLK_B), jnp.float32)
    hs = []
    for t in range(T):
        num0 = agg_ref[0:32, t, :]
        num1 = agg_ref[32:64, t, :]
        den0 = agg_ref[64, t, :]
        den1 = agg_ref[65, t, :]
        emb = 0.5 * (num0 / (den0[None, :] + 1e-16)
                     + num1 / (den1[None, :] + 1e-16)) + bgat
        emb = jnp.where(emb > 0.0, emb, jnp.exp(jnp.minimum(emb, 0.0)) - 1.0)
        gates = (jnp.dot(wih, emb, preferred_element_type=jnp.float32)
                 + jnp.dot(whh, h, preferred_element_type=jnp.float32) + bg)
        i_ = jax.nn.sigmoid(gates[0:32])
        f_ = jax.nn.sigmoid(gates[32:64])
        g_ = jnp.tanh(gates[64:96])
        o_ = jax.nn.sigmoid(gates[96:128])
        c = f_ * c + i_ * g_
        h = o_ * jnp.tanh(c)
        hs.append(h)

    wa1 = wa1_ref[...]
    ba1 = ba1_ref[...]
    wa2 = wa2_ref[...]
    ba2 = ba2_ref[...]
    scores = []
    for t in range(T):
        z = jnp.tanh(jnp.dot(wa1, hs[t], preferred_element_type=jnp.float32)
                     + ba1)
        scores.append(jnp.dot(wa2, z, preferred_element_type=jnp.float32)
                      + ba2)
    sc = jnp.concatenate(scores, axis=0)
    m = jnp.max(sc, axis=0, keepdims=True)
    ex = jnp.exp(sc - m)
    alpha = ex / jnp.sum(ex, axis=0, keepdims=True)
    h_att = alpha[0][None, :] * hs[0]
    for t in range(1, T):
        h_att = h_att + alpha[t][None, :] * hs[t]

    z = jnp.dot(wd1_ref[...], h_att, preferred_element_type=jnp.float32) \
        + bd1_ref[...]
    z = jnp.maximum(z, 0.0)
    logit = jnp.dot(wd2_ref[...], z, preferred_element_type=jnp.float32) \
        + bd2_ref[...]
    out_ref[...] = jnp.broadcast_to(logit, (8, _BLK_B))


def _tcB(agg, wih, whh, bg, wa1, ba1, wa2, ba2, wd1, bd1, wd2, bd2, bgat):
    full = lambda shape: pl.BlockSpec(shape, lambda i: tuple(0 for _ in shape))
    return pl.pallas_call(
        _tcB_body,
        grid=(N_PAD // _BLK_B,),
        in_specs=[
            pl.BlockSpec((66, T, _BLK_B), lambda i: (0, 0, i)),
            full((4 * HID, HID)),
            full((4 * HID, HID)),
            full((4 * HID, 1)),
            full((HID // 2, HID)),
            full((HID // 2, 1)),
            full((1, HID // 2)),
            full((1, 1)),
            full((HID // 2, HID)),
            full((HID // 2, 1)),
            full((1, HID // 2)),
            full((1, 1)),
            full((HID, 1)),
        ],
        out_specs=pl.BlockSpec((8, _BLK_B), lambda i: (0, i)),
        out_shape=jax.ShapeDtypeStruct((8, N_PAD), jnp.float32),
    )(agg, wih, whh, bg, wa1, ba1, wa2, ba2, wd1, bd1, wd2, bd2, bgat)


def kernel(x_sequence, edge_index, W_gat, a_src, a_dst, b_gat, W_ih, W_hh,
           b_ih, b_hh, W_att1, b_att1, W_att2, b_att2, W_dec1, b_dec1,
           W_dec2, b_dec2):
    src = jnp.concatenate([
        edge_index[0].astype(jnp.int32),
        jnp.arange(N_NODES, dtype=jnp.int32),
        jnp.full((E_PAD - E_REAL,), N_NODES, jnp.int32),
    ])
    dst = jnp.concatenate([
        edge_index[1].astype(jnp.int32),
        jnp.arange(N_NODES, dtype=jnp.int32),
        jnp.full((E_PAD - E_REAL,), N_NODES, jnp.int32),
    ])
    ep = jnp.bitwise_or(src, jnp.left_shift(dst, 14))

    A8 = jnp.zeros((8, HEADS * HID), jnp.float32)
    A8 = A8.at[0, 0:HID].set(a_src[0])
    A8 = A8.at[1, HID:2 * HID].set(a_src[1])
    A8 = A8.at[2, 0:HID].set(a_dst[0])
    A8 = A8.at[3, HID:2 * HID].set(a_dst[1])

    x2d = x_sequence.reshape(TN, F_IN)
    hpk, al8 = _tcA(x2d, W_gat[:, 0::2], W_gat[:, 1::2],
                    A8[:, 0::2], A8[:, 1::2])

    w_h = _build_sc_b0()(al8.reshape(-1), ep)
    agg = _build_sc_b1()(hpk.reshape(-1), w_h, ep)
    agg = agg.reshape(66, T, N_PAD)

    bg = (b_ih + b_hh).reshape(4 * HID, 1)
    out8 = _tcB(agg, W_ih, W_hh, bg,
                W_att1.T, b_att1.reshape(HID // 2, 1),
                W_att2.T, b_att2.reshape(1, 1),
                W_dec1.T, b_dec1.reshape(HID // 2, 1),
                W_dec2.T, b_dec2.reshape(1, 1),
                b_gat.reshape(HID, 1))
    return out8[0, :N_NODES][:, None]


# speedup vs baseline: 1.0079x; 1.0079x over previous
import functools

import jax
import jax.numpy as jnp
from jax import lax
from jax.experimental import pallas as pl
from jax.experimental.pallas import tpu as pltpu, tpu_sc as plsc

N_NODES = 10000
N_EDGES = 160000
F_IN = 128
HID = 32
HEADS = 2
T = 8

N_PAD = 10240
E_REAL = N_EDGES + N_NODES
E_PAD = 172032
EC = E_PAD // 32
NJ = EC // 16
TN = T * N_NODES


_BLK_A = 3200


def _tcA_body(x_ref, wge_ref, wgo_ref, a8e_ref, a8o_ref, pk_ref, al_ref):
    xb = x_ref[...]
    hlo = lax.dot_general(wge_ref[...], xb, (((0,), (1,)), ((), ())),
                          preferred_element_type=jnp.float32)
    hhi = lax.dot_general(wgo_ref[...], xb, (((0,), (1,)), ((), ())),
                          preferred_element_type=jnp.float32)
    al = (jnp.dot(a8e_ref[...], hlo, preferred_element_type=jnp.float32)
          + jnp.dot(a8o_ref[...], hhi,
                    preferred_element_type=jnp.float32))
    alb = lax.bitcast_convert_type(al.astype(jnp.bfloat16),
                                   jnp.uint16).astype(jnp.uint32)
    pk_s = jnp.bitwise_or(alb[0:1], jnp.left_shift(alb[1:2], 16))
    pk_d = jnp.bitwise_or(alb[2:3], jnp.left_shift(alb[3:4], 16))
    al_ref[...] = lax.bitcast_convert_type(
        jnp.concatenate([pk_s, pk_d], axis=0), jnp.int32)
    blo = lax.bitcast_convert_type(hlo.astype(jnp.bfloat16),
                                   jnp.uint16).astype(jnp.uint32)
    bhi = lax.bitcast_convert_type(hhi.astype(jnp.bfloat16),
                                   jnp.uint16).astype(jnp.uint32)
    pk_ref[...] = lax.bitcast_convert_type(
        jnp.bitwise_or(blo, jnp.left_shift(bhi, 16)), jnp.int32)


def _tcA(x2d, Wg_even, Wg_odd, A8_even, A8_odd):
    return pl.pallas_call(
        _tcA_body,
        grid=(TN // _BLK_A,),
        in_specs=[
            pl.BlockSpec((_BLK_A, F_IN), lambda i: (i, 0)),
            pl.BlockSpec((F_IN, HID), lambda i: (0, 0)),
            pl.BlockSpec((F_IN, HID), lambda i: (0, 0)),
            pl.BlockSpec((8, HID), lambda i: (0, 0)),
            pl.BlockSpec((8, HID), lambda i: (0, 0)),
        ],
        out_specs=[
            pl.BlockSpec((HID, _BLK_A), lambda i: (0, i)),
            pl.BlockSpec((2, _BLK_A), lambda i: (0, i)),
        ],
        out_shape=[
            jax.ShapeDtypeStruct((HID, TN), jnp.int32),
            jax.ShapeDtypeStruct((2, TN), jnp.int32),
        ],
    )(x2d, Wg_even, Wg_odd, A8_even, A8_odd)


def _sc_b0_body(alp, ep_h, w_h, s01, d01, epv, w0, w1):
    wid = lax.axis_index("s") * 2 + lax.axis_index("c")
    base = wid * EC
    pltpu.sync_copy(ep_h.at[pl.ds(base, EC)], epv)

    izeros16 = jnp.zeros((16,), jnp.int32)

    def zero_tail(k, _):
        off = N_NODES + k * 16
        s01[pl.ds(off, 16)] = izeros16
        d01[pl.ds(off, 16)] = izeros16
        return 0

    lax.fori_loop(0, (N_PAD - N_NODES) // 16, zero_tail, 0)

    def t_loop(t, _):
        pltpu.sync_copy(alp.at[pl.ds(0 * TN + t * N_NODES, N_NODES)],
                        s01.at[pl.ds(0, N_NODES)])
        pltpu.sync_copy(alp.at[pl.ds(1 * TN + t * N_NODES, N_NODES)],
                        d01.at[pl.ds(0, N_NODES)])

        def j_body(j):
            ev = epv[pl.ds(j * 16, 16)]
            vs = jnp.bitwise_and(ev, 0x3FFF)
            vd = lax.shift_right_logical(ev, 14)
            gs = plsc.load_gather(s01, [vs])
            gd = plsc.load_gather(d01, [vd])
            e0 = (plsc.bitcast(jnp.left_shift(gs, 16), jnp.float32)
                  + plsc.bitcast(jnp.left_shift(gd, 16), jnp.float32))
            e1 = (plsc.bitcast(jnp.bitwise_and(gs, -65536), jnp.float32)
                  + plsc.bitcast(jnp.bitwise_and(gd, -65536), jnp.float32))
            e0 = jnp.where(e0 >= 0.0, e0, 0.2 * e0)
            e1 = jnp.where(e1 >= 0.0, e1, 0.2 * e1)
            w0[pl.ds(j * 16, 16)] = jnp.exp(e0)
            w1[pl.ds(j * 16, 16)] = jnp.exp(e1)

        plsc.parallel_loop(0, NJ, 1, unroll=8)(j_body)
        pltpu.sync_copy(w0, w_h.at[pl.ds((t * 2 + 0) * E_PAD + base, EC)])
        pltpu.sync_copy(w1, w_h.at[pl.ds((t * 2 + 1) * E_PAD + base, EC)])
        return 0

    lax.fori_loop(0, T, t_loop, 0)


@functools.lru_cache(maxsize=None)
def _build_sc_b0():
    mesh = plsc.VectorSubcoreMesh(core_axis_name="c", subcore_axis_name="s")
    return pl.kernel(
        _sc_b0_body,
        mesh=mesh,
        compiler_params=pltpu.CompilerParams(needs_layout_passes=False),
        out_type=jax.ShapeDtypeStruct((T * HEADS * E_PAD,), jnp.float32),
        scratch_types=[
            pltpu.VMEM((N_PAD,), jnp.int32),
            pltpu.VMEM((N_PAD,), jnp.int32),
            pltpu.VMEM((EC,), jnp.int32),
            pltpu.VMEM((EC,), jnp.float32),
            pltpu.VMEM((EC,), jnp.float32),
        ],
    )


_SEG = N_PAD // 16


def _sc_b1_body(hpk, w_h, ep_h, agg,
                tbl, acc0, acc1, accd, ep2, w2, shr, redbuf, dout,
                semA, semB):
    cid = lax.axis_index("c")
    sid = lax.axis_index("s")
    head = cid
    pk_row = cid * 16 + sid
    col0 = 2 * pk_row
    zeros16 = jnp.zeros((16,), jnp.float32)
    izeros16 = jnp.zeros((16,), jnp.int32)

    def t_loop(t, _):
        woff = (t * 2 + head) * E_PAD

        def issue(b, par, sem):
            pltpu.async_copy(ep_h.at[pl.ds(b * EC, EC)],
                             ep2.at[pl.ds(par * EC, EC)], sem)
            pltpu.async_copy(w_h.at[pl.ds(woff + b * EC, EC)],
                             w2.at[pl.ds(par * EC, EC)], sem)

        def drain(par, sem):
            pltpu.make_async_copy(ep_h.at[pl.ds(0, EC)],
                                  ep2.at[pl.ds(par * EC, EC)], sem).wait()
            pltpu.make_async_copy(w_h.at[pl.ds(0, EC)],
                                  w2.at[pl.ds(par * EC, EC)], sem).wait()

        issue(0, 0, semA)
        pltpu.sync_copy(hpk.at[pl.ds(pk_row * TN + t * N_NODES, N_NODES)],
                        tbl.at[pl.ds(0, N_NODES)])

        def zero_acc(k):
            acc0[pl.ds(k * 16, 16)] = zeros16
            acc1[pl.ds(k * 16, 16)] = zeros16
            accd[pl.ds(k * 16, 16)] = zeros16

        plsc.parallel_loop(0, N_PAD // 16, 1, unroll=4)(zero_acc)

        def zero_tail(k):
            tbl[pl.ds(N_NODES + k * 16, 16)] = izeros16

        plsc.parallel_loop(0, (N_PAD - N_NODES) // 16, 1,
                           unroll=2)(zero_tail)

        def make_j_body(par, with_den):
            def j_body(j):
                off = par * EC + j * 16
                ev = ep2[pl.ds(off, 16)]
                vs = jnp.bitwise_and(ev, 0x3FFF)
                vd = lax.shift_right_logical(ev, 14)
                wv = w2[pl.ds(off, 16)]
                g = plsc.load_gather(tbl, [vs])
                glo = plsc.bitcast(jnp.left_shift(g, 16), jnp.float32)
                ghi = plsc.bitcast(jnp.bitwise_and(g, -65536),
                                   jnp.float32)
                plsc.addupdate_scatter(acc0, [vd], wv * glo)
                plsc.addupdate_scatter(acc1, [vd], wv * ghi)
                if with_den:
                    plsc.addupdate_scatter(accd, [vd], wv)
            return j_body

        def process(par, b):
            do_den = lax.rem(b, 16) == sid

            @pl.when(do_den)
            def _():
                plsc.parallel_loop(0, NJ, 1, unroll=8)(make_j_body(par, True))

            @pl.when(jnp.logical_not(do_den))
            def _():
                plsc.parallel_loop(0, NJ, 1,
                                   unroll=8)(make_j_body(par, False))

        def bb_loop(bb, _):
            b0 = 2 * bb
            issue(b0 + 1, 1, semB)
            drain(0, semA)
            process(0, b0)

            @pl.when(bb < 15)
            def _():
                issue(b0 + 2, 0, semA)

            drain(1, semB)
            process(1, b0 + 1)
            return 0

        lax.fori_loop(0, 16, bb_loop, 0)

        pltpu.sync_copy(acc0, agg.at[pl.ds((col0 * T + t) * N_PAD, N_PAD)])
        pltpu.sync_copy(acc1, agg.at[pl.ds(((col0 + 1) * T + t) * N_PAD,
                                           N_PAD)])

        pltpu.sync_copy(accd, shr.at[pl.ds(sid * N_PAD, N_PAD)])
        plsc.subcore_barrier()
        for p in range(16):
            pltpu.sync_copy(shr.at[pl.ds(p * N_PAD + sid * _SEG, _SEG)],
                            redbuf.at[pl.ds(p * _SEG, _SEG)])

        def red_loop(k):
            s = redbuf[pl.ds(k * 16, 16)]
            for p in range(1, 16):
                s = s + redbuf[pl.ds(p * _SEG + k * 16, 16)]
            dout[pl.ds(k * 16, 16)] = s

        plsc.parallel_loop(0, _SEG // 16, 1, unroll=2)(red_loop)
        pltpu.sync_copy(
            dout,
            agg.at[pl.ds(((64 + head) * T + t) * N_PAD + sid * _SEG, _SEG)])
        plsc.subcore_barrier()
        return 0

    lax.fori_loop(0, T, t_loop, 0)


@functools.lru_cache(maxsize=None)
def _build_sc_b1():
    mesh = plsc.VectorSubcoreMesh(core_axis_name="c", subcore_axis_name="s")
    return pl.kernel(
        _sc_b1_body,
        mesh=mesh,
        compiler_params=pltpu.CompilerParams(needs_layout_passes=False),
        out_type=jax.ShapeDtypeStruct((66 * T * N_PAD,), jnp.float32),
        scratch_types=[
            pltpu.VMEM((N_PAD,), jnp.int32),
            pltpu.VMEM((N_PAD,), jnp.float32),
            pltpu.VMEM((N_PAD,), jnp.float32),
            pltpu.VMEM((N_PAD,), jnp.float32),
            pltpu.VMEM((2 * EC,), jnp.int32),
            pltpu.VMEM((2 * EC,), jnp.float32),
            pltpu.VMEM_SHARED((16 * N_PAD,), jnp.float32),
            pltpu.VMEM((16 * _SEG,), jnp.float32),
            pltpu.VMEM((_SEG,), jnp.float32),
            pltpu.SemaphoreType.DMA,
            pltpu.SemaphoreType.DMA,
        ],
    )


_BLK_B = 1280


def _tcB_body(agg_ref, wih_ref, whh_ref, bg_ref, wa1_ref, ba1_ref, wa2_ref,
              ba2_ref, wd1_ref, bd1_ref, wd2_ref, bd2_ref, bgat_ref, out_ref):
    wih = wih_ref[...]
    whh = whh_ref[...]
    bg = bg_ref[...]
    bgat = bgat_ref[...]

    h = jnp.zeros((HID, _BLK_B), jnp.float32)
    c = jnp.zeros((HID, _BLK_B), jnp.float32)
    hs = []
    for t in range(T):
        num0 = agg_ref[0:32, t, :]
        num1 = agg_ref[32:64, t, :]
        den0 = agg_ref[64, t, :]
        den1 = agg_ref[65, t, :]
        emb = 0.5 * (num0 / (den0[None, :] + 1e-16)
                     + num1 / (den1[None, :] + 1e-16)) + bgat
        emb = jnp.where(emb > 0.0, emb, jnp.exp(jnp.minimum(emb, 0.0)) - 1.0)
        gates = (jnp.dot(wih, emb, preferred_element_type=jnp.float32)
                 + jnp.dot(whh, h, preferred_element_type=jnp.float32) + bg)
        i_ = jax.nn.sigmoid(gates[0:32])
        f_ = jax.nn.sigmoid(gates[32:64])
        g_ = jnp.tanh(gates[64:96])
        o_ = jax.nn.sigmoid(gates[96:128])
        c = f_ * c + i_ * g_
        h = o_ * jnp.tanh(c)
        hs.append(h)

    wa1 = wa1_ref[...]
    ba1 = ba1_ref[...]
    wa2 = wa2_ref[...]
    ba2 = ba2_ref[...]
    scores = []
    for t in range(T):
        z = jnp.tanh(jnp.dot(wa1, hs[t], preferred_element_type=jnp.float32)
                     + ba1)
        scores.append(jnp.dot(wa2, z, preferred_element_type=jnp.float32)
                      + ba2)
    sc = jnp.concatenate(scores, axis=0)
    m = jnp.max(sc, axis=0, keepdims=True)
    ex = jnp.exp(sc - m)
    alpha = ex / jnp.sum(ex, axis=0, keepdims=True)
    h_att = alpha[0][None, :] * hs[0]
    for t in range(1, T):
        h_att = h_att + alpha[t][None, :] * hs[t]

    z = jnp.dot(wd1_ref[...], h_att, preferred_element_type=jnp.float32) \
        + bd1_ref[...]
    z = jnp.maximum(z, 0.0)
    logit = jnp.dot(wd2_ref[...], z, preferred_element_type=jnp.float32) \
        + bd2_ref[...]
    out_ref[...] = jnp.broadcast_to(logit, (8, _BLK_B))


def _tcB(agg, wih, whh, bg, wa1, ba1, wa2, ba2, wd1, bd1, wd2, bd2, bgat):
    full = lambda shape: pl.BlockSpec(shape, lambda i: tuple(0 for _ in shape))
    return pl.pallas_call(
        _tcB_body,
        grid=(N_PAD // _BLK_B,),
        in_specs=[
            pl.BlockSpec((66, T, _BLK_B), lambda i: (0, 0, i)),
            full((4 * HID, HID)),
            full((4 * HID, HID)),
            full((4 * HID, 1)),
            full((HID // 2, HID)),
            full((HID // 2, 1)),
            full((1, HID // 2)),
            full((1, 1)),
            full((HID // 2, HID)),
            full((HID // 2, 1)),
            full((1, HID // 2)),
            full((1, 1)),
            full((HID, 1)),
        ],
        out_specs=pl.BlockSpec((8, _BLK_B), lambda i: (0, i)),
        out_shape=jax.ShapeDtypeStruct((8, N_PAD), jnp.float32),
    )(agg, wih, whh, bg, wa1, ba1, wa2, ba2, wd1, bd1, wd2, bd2, bgat)


def kernel(x_sequence, edge_index, W_gat, a_src, a_dst, b_gat, W_ih, W_hh,
           b_ih, b_hh, W_att1, b_att1, W_att2, b_att2, W_dec1, b_dec1,
           W_dec2, b_dec2):
    src = jnp.concatenate([
        edge_index[0].astype(jnp.int32),
        jnp.arange(N_NODES, dtype=jnp.int32),
        jnp.full((E_PAD - E_REAL,), N_NODES, jnp.int32),
    ])
    dst = jnp.concatenate([
        edge_index[1].astype(jnp.int32),
        jnp.arange(N_NODES, dtype=jnp.int32),
        jnp.full((E_PAD - E_REAL,), N_NODES, jnp.int32),
    ])
    ep = jnp.bitwise_or(src, jnp.left_shift(dst, 14))

    A8 = jnp.zeros((8, HEADS * HID), jnp.float32)
    A8 = A8.at[0, 0:HID].set(a_src[0])
    A8 = A8.at[1, HID:2 * HID].set(a_src[1])
    A8 = A8.at[2, 0:HID].set(a_dst[0])
    A8 = A8.at[3, HID:2 * HID].set(a_dst[1])

    x2d = x_sequence.reshape(TN, F_IN)
    hpk, al8 = _tcA(x2d, W_gat[:, 0::2], W_gat[:, 1::2],
                    A8[:, 0::2], A8[:, 1::2])

    w_h = _build_sc_b0()(al8.reshape(-1), ep)
    agg = _build_sc_b1()(hpk.reshape(-1), w_h, ep)
    agg = agg.reshape(66, T, N_PAD)

    bg = (b_ih + b_hh).reshape(4 * HID, 1)
    out8 = _tcB(agg, W_ih, W_hh, bg,
                W_att1.T, b_att1.reshape(HID // 2, 1),
                W_att2.T, b_att2.reshape(1, 1),
                W_dec1.T, b_dec1.reshape(HID // 2, 1),
                W_dec2.T, b_dec2.reshape(1, 1),
                b_gat.reshape(HID, 1))
    return out8[0, :N_NODES][:, None]
